# baseline (speedup 1.0000x reference)
import functools

import jax
import jax.numpy as jnp
import numpy as np
from jax import lax
from jax.experimental import pallas as pl
from jax.experimental.pallas import tpu as pltpu
from jax.experimental.pallas import tpu_sc as plsc

BOUND = 1.0
NUM_LEVELS = 16
LEVEL_DIM = 2
BASE_RES = 16
LOG2_HASH = 19
MAX_RES = 2048
W_MLP = 64
N_GEO = 15
OUT_CH = 1 + N_GEO
ENC_DIM = NUM_LEVELS * LEVEL_DIM
PRIME1 = np.int32(np.int64(2654435761) - (1 << 32))
PRIME2 = np.int32(805459861)
HASH_MASK = np.int32((1 << LOG2_HASH) - 1)


def _levels():
    g = np.exp((np.log(MAX_RES) - np.log(BASE_RES)) / (NUM_LEVELS - 1))
    out, off = [], 0
    for l in range(NUM_LEVELS):
        res = int(np.floor(BASE_RES * (g**l)))
        size = min((res + 1) ** 3, 2**LOG2_HASH)
        size = int(np.ceil(size / 8) * 8)
        dense = (res + 1) ** 3 <= size
        out.append(dict(res=res, size=size, off=off, dense=dense))
        off += size
    return out, off


LEVELS, TOTAL_ROWS = _levels()

NUM_CORES = 2
NUM_SUBCORES = 16
NW = NUM_CORES * NUM_SUBCORES
LANES = 16

C = 512
NBLK = C // LANES
GW = 8

NB = TOTAL_ROWS // 128
NBW = NB // NW
CB = 64
NCH = -(-NBW // CB)


def _sc_repack_body(L, t8, in_v, out_v, sem):
    wid = lax.axis_index("s") * NUM_CORES + lax.axis_index("c")
    wbase = wid * NBW
    lanes = lax.iota(jnp.int32, LANES)

    def chunk(t, _):
        boff = jnp.minimum(t * CB, NBW - CB)
        gb = wbase + boff
        pltpu.sync_copy(L.at[pl.ds(gb, CB), :, :], in_v)

        def blk_body(blk, _):
            pbase = blk * 256 + 2 * lanes
            for i in range(8):
                v0 = in_v[blk, 0, pl.ds(i * 16, LANES)]
                v1 = in_v[blk, 1, pl.ds(i * 16, LANES)]
                pv = pbase + (2 * 16 * i)
                plsc.store_scatter(
                    out_v, [lax.shift_right_logical(pv, 3), pv & 7], v0)
                pv1 = pv + 1
                plsc.store_scatter(
                    out_v, [lax.shift_right_logical(pv1, 3), pv1 & 7], v1)
            return 0

        lax.fori_loop(0, CB, blk_body, 0)
        pltpu.sync_copy(out_v, t8.at[pl.ds(gb * 32, CB * 32), :])
        return 0

    lax.fori_loop(0, NCH, chunk, 0)


@jax.jit
def _sc_repack(L):
    mesh = plsc.VectorSubcoreMesh(
        core_axis_name="c", subcore_axis_name="s",
        num_cores=NUM_CORES, num_subcores=NUM_SUBCORES)
    return pl.kernel(
        _sc_repack_body,
        out_type=jax.ShapeDtypeStruct((TOTAL_ROWS // 4, GW), jnp.float32),
        mesh=mesh,
        compiler_params=pltpu.CompilerParams(
            needs_layout_passes=False, use_tc_tiling_on_sc=False),
        scratch_types=[
            pltpu.VMEM((CB, 2, 128), jnp.float32),
            pltpu.VMEM((CB * 32, GW), jnp.float32),
            pltpu.SemaphoreType.DMA,
        ],
    )(L)


def _sc_encode_body(xyz, table, enc_hbm,
                    xyz_v, x01x, x01y, x01z,
                    fxa, fya, fza, fxb, fyb, fzb,
                    idxa, idxb, rla, rlb, valsa, valsb,
                    enc_v, sema, semb, n_points):
    pw = n_points // NW
    nchunks = pw // C
    wid = lax.axis_index("s") * NUM_CORES + lax.axis_index("c")
    wbase = wid * pw
    lanes = lax.iota(jnp.int32, LANES)
    col0 = jnp.zeros((LANES,), jnp.int32)
    bufs = [(fxa, fya, fza, idxa, rla, valsa, sema),
            (fxb, fyb, fzb, idxb, rlb, valsb, semb)]

    def run_passA(li, lv):
        fx, fy, fz, idx_v, rl_v, _, _ = bufs[li % 2]
        scale = np.float32(lv["res"] - 1)
        R = np.int32(lv["res"] + 1)
        off = np.int32(lv["off"])

        def passA(i0, scale=scale, R=R, off=off, dense=lv["dense"]):
            px = x01x[pl.ds(i0, LANES)] * scale
            py = x01y[pl.ds(i0, LANES)] * scale
            pz = x01z[pl.ds(i0, LANES)] * scale
            ix0 = px.astype(jnp.int32)
            iy0 = py.astype(jnp.int32)
            iz0 = pz.astype(jnp.int32)
            fx[pl.ds(i0, LANES)] = px - ix0.astype(jnp.float32)
            fy[pl.ds(i0, LANES)] = py - iy0.astype(jnp.float32)
            fz[pl.ds(i0, LANES)] = pz - iz0.astype(jnp.float32)
            ix1 = ix0 + 1
            iy1 = iy0 + 1
            iz1 = iz0 + 1
            if dense:
                ya = iy0 * R
                yb = iy1 * R
                za = iz0 * (R * R) + off
                zb = iz1 * (R * R) + off
            else:
                ya = iy0 * PRIME1
                yb = iy1 * PRIME1
                za = iz0 * PRIME2
                zb = iz1 * PRIME2
            for c in range(8):
                xi = ix1 if (c & 1) else ix0
                yi = yb if (c & 2) else ya
                zi = zb if (c & 4) else za
                if dense:
                    r = xi + yi + zi
                else:
                    r = ((xi ^ yi ^ zi) & HASH_MASK) + off
                e = c * C + i0
                idx_v[pl.ds(e, LANES)] = lax.shift_right_logical(r, 2)
                rl_v[pl.ds(e, LANES)] = (r & 3) * 2

        plsc.parallel_loop(0, C, LANES, unroll=2)(passA)

    def gather_copy(li):
        _, _, _, idx_v, _, vals_v, sem = bufs[li % 2]
        return pltpu.make_async_copy(table.at[idx_v], vals_v, sem)

    def run_passB(li):
        fx, fy, fz, _, rl_v, vals_v, _ = bufs[li % 2]

        def passB(i0, li=li):
            fxv = fx[pl.ds(i0, LANES)]
            fyv = fy[pl.ds(i0, LANES)]
            fzv = fz[pl.ds(i0, LANES)]
            gx = 1.0 - fxv
            gy = 1.0 - fyv
            gz = 1.0 - fzv
            rowb = i0 + lanes
            acc0 = jnp.zeros((LANES,), jnp.float32)
            acc1 = jnp.zeros((LANES,), jnp.float32)
            for c in range(8):
                wx = fxv if (c & 1) else gx
                wy = fyv if (c & 2) else gy
                wz = fzv if (c & 4) else gz
                w = (wx * wy) * wz
                rows = rowb + np.int32(c * C)
                rl = rl_v[pl.ds(c * C + i0, LANES)]
                v0 = plsc.load_gather(vals_v, [rows, rl])
                v1 = plsc.load_gather(vals_v, [rows, rl + 1])
                acc0 = acc0 + v0 * w
                acc1 = acc1 + v1 * w
            f0 = 2 * li
            f1 = 2 * li + 1
            tloc = lax.shift_right_logical(i0, 7)
            cb = i0 & 127
            enc_v[f0 >> 3, tloc, f0 & 7, pl.ds(cb, LANES)] = acc0
            enc_v[f1 >> 3, tloc, f1 & 7, pl.ds(cb, LANES)] = acc1

        plsc.parallel_loop(0, C, LANES, unroll=2)(passB)

    def chunk_body(k, _):
        pb = wbase + k * C
        pltpu.sync_copy(xyz.at[pl.ds(pb, C), :], xyz_v)

        def norm_body(i0):
            rows = i0 + lanes
            for d, ref in enumerate((x01x, x01y, x01z)):
                v = plsc.load_gather(xyz_v, [rows, col0 + d])
                v = v * (0.5 / BOUND) + 0.5
                v = jnp.minimum(jnp.maximum(v, 0.0), 1.0)
                ref[pl.ds(i0, LANES)] = v

        plsc.parallel_loop(0, C, LANES, unroll=2)(norm_body)

        run_passA(0, LEVELS[0])
        gather_copy(0).start()
        for li, lv in enumerate(LEVELS):
            if li + 1 < NUM_LEVELS:
                run_passA(li + 1, LEVELS[li + 1])
                gather_copy(li + 1).start()
            gather_copy(li).wait()
            run_passB(li)

        pltpu.sync_copy(
            enc_v, enc_hbm.at[:, pl.ds(lax.shift_right_logical(pb, 7), C // 128), :, :])
        return 0

    lax.fori_loop(0, nchunks, chunk_body, 0)


@functools.partial(jax.jit, static_argnames=("n_points",))
def _sc_encode(xyz, table, n_points):
    mesh = plsc.VectorSubcoreMesh(
        core_axis_name="c", subcore_axis_name="s",
        num_cores=NUM_CORES, num_subcores=NUM_SUBCORES)
    body = functools.partial(_sc_encode_body, n_points=n_points)
    return pl.kernel(
        body,
        out_type=jax.ShapeDtypeStruct((ENC_DIM // 8, n_points // 128, 8, 128),
                                      jnp.float32),
        mesh=mesh,
        compiler_params=pltpu.CompilerParams(
            needs_layout_passes=False, use_tc_tiling_on_sc=False),
        scratch_types=(
            [pltpu.VMEM((C, 3), jnp.float32)]
            + [pltpu.VMEM((C,), jnp.float32)] * 9
            + [pltpu.VMEM((8 * C,), jnp.int32)] * 4
            + [pltpu.VMEM((8 * C, GW), jnp.float32)] * 2
            + [pltpu.VMEM((ENC_DIM // 8, C // 128, 8, 128), jnp.float32)]
            + [pltpu.SemaphoreType.DMA] * 2
        ),
    )(xyz, table)


BM = 4096


def _mlp_body(x_ref, w0_ref, b0_ref, w1_ref, b1_ref, wo_ref, bo_ref,
              sig_ref, geo_ref):
    x = x_ref[...]
    h = jnp.maximum(
        lax.dot_general(w0_ref[...], x, (((1,), (0,)), ((), ())),
                        preferred_element_type=jnp.float32)
        + b0_ref[...], 0.0)
    h = jnp.maximum(
        lax.dot_general(w1_ref[...], h, (((1,), (0,)), ((), ())),
                        preferred_element_type=jnp.float32)
        + b1_ref[...], 0.0)
    o = (lax.dot_general(wo_ref[...], h, (((1,), (0,)), ((), ())),
                         preferred_element_type=jnp.float32)
         + bo_ref[...])
    sig_ref[...] = jnp.exp(jnp.clip(o[0:1, :], -15.0, 15.0))
    geo_ref[...] = o[1:, :]


@jax.jit
def _tc_mlp(encT, w0t, b0c, w1t, b1c, wot, boc):
    n = encT.shape[1]
    grid = (n // BM,)
    full = lambda shape: pl.BlockSpec(shape, lambda i: (0, 0))
    sig, geoT = pl.pallas_call(
        _mlp_body,
        grid=grid,
        in_specs=[
            pl.BlockSpec((ENC_DIM, BM), lambda i: (0, i)),
            full((W_MLP, ENC_DIM)),
            full((W_MLP, 1)),
            full((W_MLP, W_MLP)),
            full((W_MLP, 1)),
            full((OUT_CH, W_MLP)),
            full((OUT_CH, 1)),
        ],
        out_specs=[
            pl.BlockSpec((1, BM), lambda i: (0, i)),
            pl.BlockSpec((N_GEO, BM), lambda i: (0, i)),
        ],
        out_shape=[
            jax.ShapeDtypeStruct((1, n), jnp.float32),
            jax.ShapeDtypeStruct((N_GEO, n), jnp.float32),
        ],
    )(encT, w0t, b0c, w1t, b1c, wot, boc)
    return sig, geoT


def kernel(xyzs, table, W0, b0, W1, b1, Wout, bout):
    n = xyzs.shape[0]
    L = table.reshape(-1, 128, 2).transpose(0, 2, 1)
    table8 = _sc_repack(L)
    enc4 = _sc_encode(xyzs, table8, n)
    encT = enc4.transpose(0, 2, 1, 3).reshape(ENC_DIM, n)
    sig, geoT = _tc_mlp(encT, W0.T, b0[:, None], W1.T, b1[:, None],
                        Wout.T, bout[:, None])
    return (sig.reshape(n), geoT.T)


# speedup vs baseline: 1.2942x; 1.2942x over previous
import functools

import jax
import jax.numpy as jnp
import numpy as np
from jax import lax
from jax.experimental import pallas as pl
from jax.experimental.pallas import tpu as pltpu
from jax.experimental.pallas import tpu_sc as plsc

BOUND = 1.0
NUM_LEVELS = 16
LEVEL_DIM = 2
BASE_RES = 16
LOG2_HASH = 19
MAX_RES = 2048
W_MLP = 64
N_GEO = 15
OUT_CH = 1 + N_GEO
ENC_DIM = NUM_LEVELS * LEVEL_DIM
PRIME1 = np.int32(np.int64(2654435761) - (1 << 32))
PRIME2 = np.int32(805459861)
HASH_MASK = np.int32((1 << LOG2_HASH) - 1)


def _levels():
    g = np.exp((np.log(MAX_RES) - np.log(BASE_RES)) / (NUM_LEVELS - 1))
    out, off = [], 0
    for l in range(NUM_LEVELS):
        res = int(np.floor(BASE_RES * (g**l)))
        size = min((res + 1) ** 3, 2**LOG2_HASH)
        size = int(np.ceil(size / 8) * 8)
        dense = (res + 1) ** 3 <= size
        out.append(dict(res=res, size=size, off=off, dense=dense))
        off += size
    return out, off


LEVELS, TOTAL_ROWS = _levels()

NUM_CORES = 2
NUM_SUBCORES = 16
NW = NUM_CORES * NUM_SUBCORES
LANES = 16

C = 256
NBLK = C // LANES
GW = 8
N_RES = 2
RES_GRAN = LEVELS[N_RES]["off"] // 4

NB = TOTAL_ROWS // 128
NBW = NB // NW
CB = 64
NCH = -(-NBW // CB)


def _sc_repack_body(L, t8, in_v, out_v, sem):
    wid = lax.axis_index("s") * NUM_CORES + lax.axis_index("c")
    wbase = wid * NBW
    lanes = lax.iota(jnp.int32, LANES)

    def chunk(t, _):
        boff = jnp.minimum(t * CB, NBW - CB)
        gb = wbase + boff
        pltpu.sync_copy(L.at[pl.ds(gb, CB), :, :], in_v)

        def blk_body(blk, _):
            pbase = blk * 256 + 2 * lanes
            for i in range(8):
                v0 = in_v[blk, 0, pl.ds(i * 16, LANES)]
                v1 = in_v[blk, 1, pl.ds(i * 16, LANES)]
                pv = pbase + (2 * 16 * i)
                plsc.store_scatter(
                    out_v, [lax.shift_right_logical(pv, 3), pv & 7], v0)
                pv1 = pv + 1
                plsc.store_scatter(
                    out_v, [lax.shift_right_logical(pv1, 3), pv1 & 7], v1)
            return 0

        lax.fori_loop(0, CB, blk_body, 0)
        pltpu.sync_copy(out_v, t8.at[pl.ds(gb * 32, CB * 32), :])
        return 0

    lax.fori_loop(0, NCH, chunk, 0)


@jax.jit
def _sc_repack(L):
    mesh = plsc.VectorSubcoreMesh(
        core_axis_name="c", subcore_axis_name="s",
        num_cores=NUM_CORES, num_subcores=NUM_SUBCORES)
    return pl.kernel(
        _sc_repack_body,
        out_type=jax.ShapeDtypeStruct((TOTAL_ROWS // 4, GW), jnp.float32),
        mesh=mesh,
        compiler_params=pltpu.CompilerParams(
            needs_layout_passes=False, use_tc_tiling_on_sc=False),
        scratch_types=[
            pltpu.VMEM((CB, 2, 128), jnp.float32),
            pltpu.VMEM((CB * 32, GW), jnp.float32),
            pltpu.SemaphoreType.DMA,
        ],
    )(L)


def _sc_encode_body(xyz, table, enc_hbm,
                    xyz_v, x01x, x01y, x01z,
                    fxa, fya, fza, fxb, fyb, fzb,
                    idxa, idxb, rla, rlb, valsa, valsb,
                    enc_v, tab01_v, sema, semb, n_points):
    pw = n_points // NW
    nchunks = pw // C
    wid = lax.axis_index("s") * NUM_CORES + lax.axis_index("c")
    wbase = wid * pw
    lanes = lax.iota(jnp.int32, LANES)
    col0 = jnp.zeros((LANES,), jnp.int32)
    bufs = [(fxa, fya, fza, idxa, rla, valsa, sema),
            (fxb, fyb, fzb, idxb, rlb, valsb, semb)]

    def run_passA(li, lv):
        fx, fy, fz, idx_v, rl_v, _, _ = bufs[li % 2]
        scale = np.float32(lv["res"] - 1)
        R = np.int32(lv["res"] + 1)
        off = np.int32(lv["off"])

        def passA(i0, scale=scale, R=R, off=off, dense=lv["dense"]):
            px = x01x[pl.ds(i0, LANES)] * scale
            py = x01y[pl.ds(i0, LANES)] * scale
            pz = x01z[pl.ds(i0, LANES)] * scale
            ix0 = px.astype(jnp.int32)
            iy0 = py.astype(jnp.int32)
            iz0 = pz.astype(jnp.int32)
            fx[pl.ds(i0, LANES)] = px - ix0.astype(jnp.float32)
            fy[pl.ds(i0, LANES)] = py - iy0.astype(jnp.float32)
            fz[pl.ds(i0, LANES)] = pz - iz0.astype(jnp.float32)
            ix1 = ix0 + 1
            iy1 = iy0 + 1
            iz1 = iz0 + 1
            if dense:
                ya = iy0 * R
                yb = iy1 * R
                za = iz0 * (R * R) + off
                zb = iz1 * (R * R) + off
            else:
                ya = iy0 * PRIME1
                yb = iy1 * PRIME1
                za = iz0 * PRIME2
                zb = iz1 * PRIME2
            for c in range(8):
                xi = ix1 if (c & 1) else ix0
                yi = yb if (c & 2) else ya
                zi = zb if (c & 4) else za
                if dense:
                    r = xi + yi + zi
                else:
                    r = ((xi ^ yi ^ zi) & HASH_MASK) + off
                e = c * C + i0
                idx_v[pl.ds(e, LANES)] = lax.shift_right_logical(r, 2)
                rl_v[pl.ds(e, LANES)] = (r & 3) * 2

        plsc.parallel_loop(0, C, LANES, unroll=2)(passA)

    def gather_copy(li):
        _, _, _, idx_v, _, vals_v, sem = bufs[li % 2]
        return pltpu.make_async_copy(table.at[idx_v], vals_v, sem)

    def run_passB(li):
        fx, fy, fz, idx_v, rl_v, vals_v, _ = bufs[li % 2]
        resident = li < N_RES

        def passB(i0, li=li):
            fxv = fx[pl.ds(i0, LANES)]
            fyv = fy[pl.ds(i0, LANES)]
            fzv = fz[pl.ds(i0, LANES)]
            gx = 1.0 - fxv
            gy = 1.0 - fyv
            gz = 1.0 - fzv
            rowb = i0 + lanes
            acc0 = jnp.zeros((LANES,), jnp.float32)
            acc1 = jnp.zeros((LANES,), jnp.float32)
            for c in range(8):
                wx = fxv if (c & 1) else gx
                wy = fyv if (c & 2) else gy
                wz = fzv if (c & 4) else gz
                w = (wx * wy) * wz
                rl = rl_v[pl.ds(c * C + i0, LANES)]
                if resident:
                    rows = idx_v[pl.ds(c * C + i0, LANES)]
                    src = tab01_v
                else:
                    rows = rowb + np.int32(c * C)
                    src = vals_v
                v0 = plsc.load_gather(src, [rows, rl])
                v1 = plsc.load_gather(src, [rows, rl + 1])
                acc0 = acc0 + v0 * w
                acc1 = acc1 + v1 * w
            f0 = 2 * li
            f1 = 2 * li + 1
            tloc = lax.shift_right_logical(i0, 7)
            cb = i0 & 127
            enc_v[f0 >> 3, tloc, f0 & 7, pl.ds(cb, LANES)] = acc0
            enc_v[f1 >> 3, tloc, f1 & 7, pl.ds(cb, LANES)] = acc1

        plsc.parallel_loop(0, C, LANES, unroll=2)(passB)

    pltpu.sync_copy(table.at[pl.ds(0, RES_GRAN), :], tab01_v)

    def chunk_body(k, _):
        pb = wbase + k * C
        pltpu.sync_copy(xyz.at[pl.ds(pb, C), :], xyz_v)

        def norm_body(i0):
            rows = i0 + lanes
            for d, ref in enumerate((x01x, x01y, x01z)):
                v = plsc.load_gather(xyz_v, [rows, col0 + d])
                v = v * (0.5 / BOUND) + 0.5
                v = jnp.minimum(jnp.maximum(v, 0.0), 1.0)
                ref[pl.ds(i0, LANES)] = v

        plsc.parallel_loop(0, C, LANES, unroll=2)(norm_body)

        for li in range(N_RES):
            run_passA(li, LEVELS[li])
            run_passB(li)

        run_passA(N_RES, LEVELS[N_RES])
        gather_copy(N_RES).start()
        for li in range(N_RES, NUM_LEVELS):
            if li + 1 < NUM_LEVELS:
                run_passA(li + 1, LEVELS[li + 1])
                gather_copy(li + 1).start()
            gather_copy(li).wait()
            run_passB(li)

        pltpu.sync_copy(
            enc_v, enc_hbm.at[:, pl.ds(lax.shift_right_logical(pb, 7), C // 128), :, :])
        return 0

    lax.fori_loop(0, nchunks, chunk_body, 0)


@functools.partial(jax.jit, static_argnames=("n_points",))
def _sc_encode(xyz, table, n_points):
    mesh = plsc.VectorSubcoreMesh(
        core_axis_name="c", subcore_axis_name="s",
        num_cores=NUM_CORES, num_subcores=NUM_SUBCORES)
    body = functools.partial(_sc_encode_body, n_points=n_points)
    return pl.kernel(
        body,
        out_type=jax.ShapeDtypeStruct((ENC_DIM // 8, n_points // 128, 8, 128),
                                      jnp.float32),
        mesh=mesh,
        compiler_params=pltpu.CompilerParams(
            needs_layout_passes=False, use_tc_tiling_on_sc=False),
        scratch_types=(
            [pltpu.VMEM((C, 3), jnp.float32)]
            + [pltpu.VMEM((C,), jnp.float32)] * 9
            + [pltpu.VMEM((8 * C,), jnp.int32)] * 4
            + [pltpu.VMEM((8 * C, GW), jnp.float32)] * 2
            + [pltpu.VMEM((ENC_DIM // 8, C // 128, 8, 128), jnp.float32)]
            + [pltpu.VMEM((RES_GRAN, GW), jnp.float32)]
            + [pltpu.SemaphoreType.DMA] * 2
        ),
    )(xyz, table)


BM = 4096


def _mlp_body(x_ref, w0_ref, b0_ref, w1_ref, b1_ref, wo_ref, bo_ref,
              sig_ref, geo_ref):
    x = x_ref[...]
    h = jnp.maximum(
        lax.dot_general(w0_ref[...], x, (((1,), (0,)), ((), ())),
                        preferred_element_type=jnp.float32)
        + b0_ref[...], 0.0)
    h = jnp.maximum(
        lax.dot_general(w1_ref[...], h, (((1,), (0,)), ((), ())),
                        preferred_element_type=jnp.float32)
        + b1_ref[...], 0.0)
    o = (lax.dot_general(wo_ref[...], h, (((1,), (0,)), ((), ())),
                         preferred_element_type=jnp.float32)
         + bo_ref[...])
    sig_ref[...] = jnp.exp(jnp.clip(o[0:1, :], -15.0, 15.0))
    geo_ref[...] = o[1:, :]


@jax.jit
def _tc_mlp(encT, w0t, b0c, w1t, b1c, wot, boc):
    n = encT.shape[1]
    grid = (n // BM,)
    full = lambda shape: pl.BlockSpec(shape, lambda i: (0, 0))
    sig, geoT = pl.pallas_call(
        _mlp_body,
        grid=grid,
        in_specs=[
            pl.BlockSpec((ENC_DIM, BM), lambda i: (0, i)),
            full((W_MLP, ENC_DIM)),
            full((W_MLP, 1)),
            full((W_MLP, W_MLP)),
            full((W_MLP, 1)),
            full((OUT_CH, W_MLP)),
            full((OUT_CH, 1)),
        ],
        out_specs=[
            pl.BlockSpec((1, BM), lambda i: (0, i)),
            pl.BlockSpec((N_GEO, BM), lambda i: (0, i)),
        ],
        out_shape=[
            jax.ShapeDtypeStruct((1, n), jnp.float32),
            jax.ShapeDtypeStruct((N_GEO, n), jnp.float32),
        ],
    )(encT, w0t, b0c, w1t, b1c, wot, boc)
    return sig, geoT


def kernel(xyzs, table, W0, b0, W1, b1, Wout, bout):
    n = xyzs.shape[0]
    L = table.reshape(-1, 128, 2).transpose(0, 2, 1)
    table8 = _sc_repack(L)
    enc4 = _sc_encode(xyzs, table8, n)
    encT = enc4.transpose(0, 2, 1, 3).reshape(ENC_DIM, n)
    sig, geoT = _tc_mlp(encT, W0.T, b0[:, None], W1.T, b1[:, None],
                        Wout.T, bout[:, None])
    return (sig.reshape(n), geoT.T)


# speedup vs baseline: 1.4194x; 1.0968x over previous
import functools

import jax
import jax.numpy as jnp
import numpy as np
from jax import lax
from jax.experimental import pallas as pl
from jax.experimental.pallas import tpu as pltpu
from jax.experimental.pallas import tpu_sc as plsc

BOUND = 1.0
NUM_LEVELS = 16
LEVEL_DIM = 2
BASE_RES = 16
LOG2_HASH = 19
MAX_RES = 2048
W_MLP = 64
N_GEO = 15
OUT_CH = 1 + N_GEO
ENC_DIM = NUM_LEVELS * LEVEL_DIM
PRIME1 = np.int32(np.int64(2654435761) - (1 << 32))
PRIME2 = np.int32(805459861)
HASH_MASK = np.int32((1 << LOG2_HASH) - 1)


def _levels():
    g = np.exp((np.log(MAX_RES) - np.log(BASE_RES)) / (NUM_LEVELS - 1))
    out, off = [], 0
    for l in range(NUM_LEVELS):
        res = int(np.floor(BASE_RES * (g**l)))
        size = min((res + 1) ** 3, 2**LOG2_HASH)
        size = int(np.ceil(size / 8) * 8)
        dense = (res + 1) ** 3 <= size
        out.append(dict(res=res, size=size, off=off, dense=dense))
        off += size
    return out, off


LEVELS, TOTAL_ROWS = _levels()

NUM_CORES = 2
NUM_SUBCORES = 16
NW = NUM_CORES * NUM_SUBCORES
LANES = 16

C = 256
NBLK = C // LANES
GW = 8
N_RES = 2
RES_GRAN = LEVELS[N_RES]["off"] // 4
N_SH = 5
SH_GRAN = (LEVELS[N_SH]["off"] - LEVELS[N_RES]["off"]) // 4
SH_PER_TILE = SH_GRAN // NUM_SUBCORES

NB = TOTAL_ROWS // 128
NBW = NB // NW
CB = 64
NCH = -(-NBW // CB)


def _sc_repack_body(L, t8, in_v, out_v, sem):
    wid = lax.axis_index("s") * NUM_CORES + lax.axis_index("c")
    wbase = wid * NBW
    lanes = lax.iota(jnp.int32, LANES)

    def chunk(t, _):
        boff = jnp.minimum(t * CB, NBW - CB)
        gb = wbase + boff
        pltpu.sync_copy(L.at[pl.ds(gb, CB), :, :], in_v)

        def blk_body(blk, _):
            pbase = blk * 256 + 2 * lanes
            for i in range(8):
                v0 = in_v[blk, 0, pl.ds(i * 16, LANES)]
                v1 = in_v[blk, 1, pl.ds(i * 16, LANES)]
                pv = pbase + (2 * 16 * i)
                plsc.store_scatter(
                    out_v, [lax.shift_right_logical(pv, 3), pv & 7], v0)
                pv1 = pv + 1
                plsc.store_scatter(
                    out_v, [lax.shift_right_logical(pv1, 3), pv1 & 7], v1)
            return 0

        lax.fori_loop(0, CB, blk_body, 0)
        pltpu.sync_copy(out_v, t8.at[pl.ds(gb * 32, CB * 32), :])
        return 0

    lax.fori_loop(0, NCH, chunk, 0)


@jax.jit
def _sc_repack(L):
    mesh = plsc.VectorSubcoreMesh(
        core_axis_name="c", subcore_axis_name="s",
        num_cores=NUM_CORES, num_subcores=NUM_SUBCORES)
    return pl.kernel(
        _sc_repack_body,
        out_type=jax.ShapeDtypeStruct((TOTAL_ROWS // 4, GW), jnp.float32),
        mesh=mesh,
        compiler_params=pltpu.CompilerParams(
            needs_layout_passes=False, use_tc_tiling_on_sc=False),
        scratch_types=[
            pltpu.VMEM((CB, 2, 128), jnp.float32),
            pltpu.VMEM((CB * 32, GW), jnp.float32),
            pltpu.SemaphoreType.DMA,
        ],
    )(L)


def _sc_encode_body(xyz, table, enc_hbm,
                    xyz_v, x01x, x01y, x01z,
                    fxa, fya, fza, fxb, fyb, fzb,
                    idxa, idxb, rla, rlb, valsa, valsb,
                    enc_v, tab01_v, tabsh_v, sema, semb, n_points):
    pw = n_points // NW
    nchunks = pw // C
    wid = lax.axis_index("s") * NUM_CORES + lax.axis_index("c")
    wbase = wid * pw
    lanes = lax.iota(jnp.int32, LANES)
    col0 = jnp.zeros((LANES,), jnp.int32)
    bufs = [(fxa, fya, fza, idxa, rla, valsa, sema),
            (fxb, fyb, fzb, idxb, rlb, valsb, semb)]

    def run_passA(li, lv):
        fx, fy, fz, idx_v, rl_v, _, _ = bufs[li % 2]
        scale = np.float32(lv["res"] - 1)
        R = np.int32(lv["res"] + 1)
        off = np.int32(lv["off"])

        def passA(i0, scale=scale, R=R, off=off, dense=lv["dense"]):
            px = x01x[pl.ds(i0, LANES)] * scale
            py = x01y[pl.ds(i0, LANES)] * scale
            pz = x01z[pl.ds(i0, LANES)] * scale
            ix0 = px.astype(jnp.int32)
            iy0 = py.astype(jnp.int32)
            iz0 = pz.astype(jnp.int32)
            fx[pl.ds(i0, LANES)] = px - ix0.astype(jnp.float32)
            fy[pl.ds(i0, LANES)] = py - iy0.astype(jnp.float32)
            fz[pl.ds(i0, LANES)] = pz - iz0.astype(jnp.float32)
            ix1 = ix0 + 1
            iy1 = iy0 + 1
            iz1 = iz0 + 1
            if dense:
                ya = iy0 * R
                yb = iy1 * R
                za = iz0 * (R * R) + off
                zb = iz1 * (R * R) + off
            else:
                ya = iy0 * PRIME1
                yb = iy1 * PRIME1
                za = iz0 * PRIME2
                zb = iz1 * PRIME2
            gsub = np.int32(RES_GRAN if N_RES <= li < N_SH else 0)
            for c in range(8):
                xi = ix1 if (c & 1) else ix0
                yi = yb if (c & 2) else ya
                zi = zb if (c & 4) else za
                if dense:
                    r = xi + yi + zi
                else:
                    r = ((xi ^ yi ^ zi) & HASH_MASK) + off
                e = c * C + i0
                idx_v[pl.ds(e, LANES)] = lax.shift_right_logical(r, 2) - gsub
                rl_v[pl.ds(e, LANES)] = (r & 3) * 2

        plsc.parallel_loop(0, C, LANES, unroll=2)(passA)

    def gather_copy(li):
        _, _, _, idx_v, _, vals_v, sem = bufs[li % 2]
        src = tabsh_v if N_RES <= li < N_SH else table
        return pltpu.make_async_copy(src.at[idx_v], vals_v, sem)

    def run_passB(li):
        fx, fy, fz, idx_v, rl_v, vals_v, _ = bufs[li % 2]
        resident = li < N_RES

        def passB(i0, li=li):
            fxv = fx[pl.ds(i0, LANES)]
            fyv = fy[pl.ds(i0, LANES)]
            fzv = fz[pl.ds(i0, LANES)]
            gx = 1.0 - fxv
            gy = 1.0 - fyv
            gz = 1.0 - fzv
            rowb = i0 + lanes
            acc0 = jnp.zeros((LANES,), jnp.float32)
            acc1 = jnp.zeros((LANES,), jnp.float32)
            for c in range(8):
                wx = fxv if (c & 1) else gx
                wy = fyv if (c & 2) else gy
                wz = fzv if (c & 4) else gz
                w = (wx * wy) * wz
                rl = rl_v[pl.ds(c * C + i0, LANES)]
                if resident:
                    rows = idx_v[pl.ds(c * C + i0, LANES)]
                    src = tab01_v
                else:
                    rows = rowb + np.int32(c * C)
                    src = vals_v
                v0 = plsc.load_gather(src, [rows, rl])
                v1 = plsc.load_gather(src, [rows, rl + 1])
                acc0 = acc0 + v0 * w
                acc1 = acc1 + v1 * w
            f0 = 2 * li
            f1 = 2 * li + 1
            tloc = lax.shift_right_logical(i0, 7)
            cb = i0 & 127
            enc_v[f0 >> 3, tloc, f0 & 7, pl.ds(cb, LANES)] = acc0
            enc_v[f1 >> 3, tloc, f1 & 7, pl.ds(cb, LANES)] = acc1

        plsc.parallel_loop(0, C, LANES, unroll=2)(passB)

    pltpu.sync_copy(table.at[pl.ds(0, RES_GRAN), :], tab01_v)
    sid = lax.axis_index("s")
    pltpu.sync_copy(
        table.at[pl.ds(RES_GRAN + sid * SH_PER_TILE, SH_PER_TILE), :],
        tabsh_v.at[pl.ds(sid * SH_PER_TILE, SH_PER_TILE), :])
    plsc.subcore_barrier()

    def chunk_body(k, _):
        pb = wbase + k * C
        pltpu.sync_copy(xyz.at[pl.ds(pb, C), :], xyz_v)

        def norm_body(i0):
            rows = i0 + lanes
            for d, ref in enumerate((x01x, x01y, x01z)):
                v = plsc.load_gather(xyz_v, [rows, col0 + d])
                v = v * (0.5 / BOUND) + 0.5
                v = jnp.minimum(jnp.maximum(v, 0.0), 1.0)
                ref[pl.ds(i0, LANES)] = v

        plsc.parallel_loop(0, C, LANES, unroll=2)(norm_body)

        for li in range(N_RES):
            run_passA(li, LEVELS[li])
            run_passB(li)

        run_passA(N_RES, LEVELS[N_RES])
        gather_copy(N_RES).start()
        for li in range(N_RES, NUM_LEVELS):
            if li + 1 < NUM_LEVELS:
                run_passA(li + 1, LEVELS[li + 1])
                gather_copy(li + 1).start()
            gather_copy(li).wait()
            run_passB(li)

        pltpu.sync_copy(
            enc_v, enc_hbm.at[:, pl.ds(lax.shift_right_logical(pb, 7), C // 128), :, :])
        return 0

    lax.fori_loop(0, nchunks, chunk_body, 0)


@functools.partial(jax.jit, static_argnames=("n_points",))
def _sc_encode(xyz, table, n_points):
    mesh = plsc.VectorSubcoreMesh(
        core_axis_name="c", subcore_axis_name="s",
        num_cores=NUM_CORES, num_subcores=NUM_SUBCORES)
    body = functools.partial(_sc_encode_body, n_points=n_points)
    return pl.kernel(
        body,
        out_type=jax.ShapeDtypeStruct((ENC_DIM // 8, n_points // 128, 8, 128),
                                      jnp.float32),
        mesh=mesh,
        compiler_params=pltpu.CompilerParams(
            needs_layout_passes=False, use_tc_tiling_on_sc=False),
        scratch_types=(
            [pltpu.VMEM((C, 3), jnp.float32)]
            + [pltpu.VMEM((C,), jnp.float32)] * 9
            + [pltpu.VMEM((8 * C,), jnp.int32)] * 4
            + [pltpu.VMEM((8 * C, GW), jnp.float32)] * 2
            + [pltpu.VMEM((ENC_DIM // 8, C // 128, 8, 128), jnp.float32)]
            + [pltpu.VMEM((RES_GRAN, GW), jnp.float32)]
            + [pltpu.VMEM_SHARED((SH_GRAN, GW), jnp.float32)]
            + [pltpu.SemaphoreType.DMA] * 2
        ),
    )(xyz, table)


BM = 4096


def _mlp_body(x_ref, w0_ref, b0_ref, w1_ref, b1_ref, wo_ref, bo_ref,
              sig_ref, geo_ref):
    x = x_ref[...]
    h = jnp.maximum(
        lax.dot_general(w0_ref[...], x, (((1,), (0,)), ((), ())),
                        preferred_element_type=jnp.float32)
        + b0_ref[...], 0.0)
    h = jnp.maximum(
        lax.dot_general(w1_ref[...], h, (((1,), (0,)), ((), ())),
                        preferred_element_type=jnp.float32)
        + b1_ref[...], 0.0)
    o = (lax.dot_general(wo_ref[...], h, (((1,), (0,)), ((), ())),
                         preferred_element_type=jnp.float32)
         + bo_ref[...])
    sig_ref[...] = jnp.exp(jnp.clip(o[0:1, :], -15.0, 15.0))
    geo_ref[...] = o[1:, :]


@jax.jit
def _tc_mlp(encT, w0t, b0c, w1t, b1c, wot, boc):
    n = encT.shape[1]
    grid = (n // BM,)
    full = lambda shape: pl.BlockSpec(shape, lambda i: (0, 0))
    sig, geoT = pl.pallas_call(
        _mlp_body,
        grid=grid,
        in_specs=[
            pl.BlockSpec((ENC_DIM, BM), lambda i: (0, i)),
            full((W_MLP, ENC_DIM)),
            full((W_MLP, 1)),
            full((W_MLP, W_MLP)),
            full((W_MLP, 1)),
            full((OUT_CH, W_MLP)),
            full((OUT_CH, 1)),
        ],
        out_specs=[
            pl.BlockSpec((1, BM), lambda i: (0, i)),
            pl.BlockSpec((N_GEO, BM), lambda i: (0, i)),
        ],
        out_shape=[
            jax.ShapeDtypeStruct((1, n), jnp.float32),
            jax.ShapeDtypeStruct((N_GEO, n), jnp.float32),
        ],
    )(encT, w0t, b0c, w1t, b1c, wot, boc)
    return sig, geoT


def kernel(xyzs, table, W0, b0, W1, b1, Wout, bout):
    n = xyzs.shape[0]
    L = table.reshape(-1, 128, 2).transpose(0, 2, 1)
    table8 = _sc_repack(L)
    enc4 = _sc_encode(xyzs, table8, n)
    encT = enc4.transpose(0, 2, 1, 3).reshape(ENC_DIM, n)
    sig, geoT = _tc_mlp(encT, W0.T, b0[:, None], W1.T, b1[:, None],
                        Wout.T, bout[:, None])
    return (sig.reshape(n), geoT.T)
